# Initial kernel scaffold; baseline (speedup 1.0000x reference)
import jax
import jax.numpy as jnp
from jax.experimental import pallas as pl


def kernel(x, edge_index, W, b):
    raise NotImplementedError("write your pallas kernel here")


# speedup vs baseline: 17.8779x; 17.8779x over previous
import functools

import jax
import jax.numpy as jnp
from jax import lax
from jax.experimental import pallas as pl
from jax.experimental.pallas import tpu as pltpu
from jax.experimental.pallas import tpu_sc as plsc

N_NODES = 10000
N_PAD = 10240
N_EDGES = 320000
D = 128

NC, NS = 2, 16
NW = NC * NS
EPW = N_EDGES // NW
CHUNK = 80
NCHUNK = EPW // CHUNK
RPT = N_PAD // NS

_mesh = plsc.VectorSubcoreMesh(core_axis_name="c", subcore_axis_name="s")


@functools.partial(
    pl.kernel,
    mesh=_mesh,
    out_type=jax.ShapeDtypeStruct((NC * N_PAD,), jnp.float32),
    scratch_types=[
        pltpu.VMEM((CHUNK,), jnp.int32),
        pltpu.VMEM((CHUNK,), jnp.float32),
        pltpu.VMEM((RPT,), jnp.float32),
        pltpu.VMEM_SHARED((N_PAD,), jnp.float32),
    ],
)
def _deg_kernel(dst_hbm, degp_hbm, didx_v, ones_v, zb_v, deg_sh):
    c = lax.axis_index("c")
    s = lax.axis_index("s")
    base = (c * NS + s) * EPW
    row = pl.multiple_of(s * RPT, 8)
    for i in range(CHUNK // 16):
        ones_v[pl.ds(i * 16, 16)] = jnp.ones((16,), jnp.float32)
    for i in range(RPT // 16):
        zb_v[pl.ds(i * 16, 16)] = jnp.zeros((16,), jnp.float32)
    pltpu.sync_copy(zb_v, deg_sh.at[pl.ds(row, RPT)])
    plsc.subcore_barrier()

    def body(k, _):
        off = pl.multiple_of(base + k * CHUNK, 8)
        pltpu.sync_copy(dst_hbm.at[pl.ds(off, CHUNK)], didx_v)
        pltpu.sync_copy(ones_v, deg_sh.at[didx_v], add=True)
        return _

    lax.fori_loop(0, NCHUNK, body, None)
    plsc.subcore_barrier()
    pltpu.sync_copy(deg_sh.at[pl.ds(row, RPT)], zb_v)
    pltpu.sync_copy(zb_v,
                    degp_hbm.at[pl.ds(pl.multiple_of(c * N_PAD + s * RPT, 8),
                                      RPT)])


def _scale_body(dpT_ref, x_ref, w_ref, y_ref, dinv_ref):
    deg = dpT_ref[:, 0:1] + dpT_ref[:, 1:2] + 1.0
    dinv = lax.rsqrt(deg)
    dinv_ref[...] = dinv
    y_ref[...] = dinv * jnp.dot(x_ref[...], w_ref[...],
                                preferred_element_type=jnp.float32)


_scale_call = pl.pallas_call(
    _scale_body,
    out_shape=[
        jax.ShapeDtypeStruct((N_PAD, D), jnp.float32),
        jax.ShapeDtypeStruct((N_PAD, 1), jnp.float32),
    ],
)


@functools.partial(
    pl.kernel,
    mesh=_mesh,
    out_type=jax.ShapeDtypeStruct((NC * N_PAD, D), jnp.float32),
    scratch_types=[
        pltpu.VMEM((CHUNK,), jnp.int32),
        pltpu.VMEM((CHUNK,), jnp.int32),
        pltpu.VMEM((CHUNK, D), jnp.float32),
        pltpu.SemaphoreType.DMA,
        pltpu.VMEM_SHARED((N_PAD, D), jnp.float32),
    ],
)
def _agg_kernel(src_hbm, dst_hbm, y_hbm, p_hbm,
                sidx_v, didx_v, rows_v, sem, agg_sh):
    c = lax.axis_index("c")
    s = lax.axis_index("s")
    base = (c * NS + s) * EPW
    row = pl.multiple_of(s * RPT, 8)
    for i in range(CHUNK):
        for j in range(D // 16):
            rows_v[i, pl.ds(j * 16, 16)] = jnp.zeros((16,), jnp.float32)
    for j in range(RPT // CHUNK):
        pltpu.sync_copy(rows_v, agg_sh.at[pl.ds(row + j * CHUNK, CHUNK)])
    plsc.subcore_barrier()

    def body(k, _):
        off = pl.multiple_of(base + k * CHUNK, 8)
        pltpu.sync_copy(src_hbm.at[pl.ds(off, CHUNK)], sidx_v)
        pltpu.sync_copy(dst_hbm.at[pl.ds(off, CHUNK)], didx_v)
        pltpu.async_copy(y_hbm.at[sidx_v], rows_v, sem).wait()
        pltpu.sync_copy(rows_v, agg_sh.at[didx_v], add=True)
        return _

    lax.fori_loop(0, NCHUNK, body, None)
    plsc.subcore_barrier()
    gbase = pl.multiple_of(c * N_PAD + s * RPT, 8)
    for j in range(RPT // CHUNK):
        pltpu.sync_copy(agg_sh.at[pl.ds(row + j * CHUNK, CHUNK)], rows_v)
        pltpu.sync_copy(rows_v, p_hbm.at[pl.ds(gbase + j * CHUNK, CHUNK)])


def _combine_body(p_ref, y_ref, dinv_ref, b_ref, o_ref):
    o_ref[...] = dinv_ref[...] * (p_ref[:N_PAD] + p_ref[N_PAD:] + y_ref[...]) \
        + b_ref[...]


_combine_call = pl.pallas_call(
    _combine_body,
    out_shape=jax.ShapeDtypeStruct((N_PAD, D), jnp.float32),
)


def kernel(x, edge_index, W, b):
    ei = edge_index.astype(jnp.int32)
    src, dst = ei[0], ei[1]
    xp = jnp.pad(x, ((0, N_PAD - N_NODES), (0, 0)))

    degp = _deg_kernel(dst)
    y, dinv = _scale_call(degp.reshape(NC, N_PAD).T, xp, W)
    p = _agg_kernel(src, dst, y)
    out = _combine_call(p, y, dinv, b.reshape(1, D))
    return out[:N_NODES]


# speedup vs baseline: 26.8095x; 1.4996x over previous
import functools

import jax
import jax.numpy as jnp
from jax import lax
from jax.experimental import pallas as pl
from jax.experimental.pallas import tpu as pltpu
from jax.experimental.pallas import tpu_sc as plsc

N_NODES = 10000
N_PAD = 10240
N_EDGES = 320000
D = 128

NC, NS = 2, 16
NW = NC * NS
EPW = N_EDGES // NW
CHUNK = 200
NCHUNK = EPW // CHUNK
RPT = N_PAD // NS

_mesh = plsc.VectorSubcoreMesh(core_axis_name="c", subcore_axis_name="s")


@functools.partial(
    pl.kernel,
    mesh=_mesh,
    out_type=jax.ShapeDtypeStruct((NC * N_PAD,), jnp.float32),
    scratch_types=[
        pltpu.VMEM((CHUNK,), jnp.int32),
        pltpu.VMEM((CHUNK,), jnp.float32),
        pltpu.VMEM((RPT,), jnp.float32),
        pltpu.VMEM_SHARED((N_PAD,), jnp.float32),
    ],
)
def _deg_kernel(dst_hbm, degp_hbm, didx_v, ones_v, zb_v, deg_sh):
    c = lax.axis_index("c")
    s = lax.axis_index("s")
    base = (c * NS + s) * EPW
    row = pl.multiple_of(s * RPT, 8)
    for i in range(CHUNK // 16):
        ones_v[pl.ds(i * 16, 16)] = jnp.ones((16,), jnp.float32)
    for i in range(RPT // 16):
        zb_v[pl.ds(i * 16, 16)] = jnp.zeros((16,), jnp.float32)
    pltpu.sync_copy(zb_v, deg_sh.at[pl.ds(row, RPT)])
    plsc.subcore_barrier()

    def body(k, _):
        off = pl.multiple_of(base + k * CHUNK, 8)
        pltpu.sync_copy(dst_hbm.at[pl.ds(off, CHUNK)], didx_v)
        pltpu.sync_copy(ones_v, deg_sh.at[didx_v], add=True)
        return _

    lax.fori_loop(0, NCHUNK, body, None)
    plsc.subcore_barrier()
    pltpu.sync_copy(deg_sh.at[pl.ds(row, RPT)], zb_v)
    pltpu.sync_copy(zb_v,
                    degp_hbm.at[pl.ds(pl.multiple_of(c * N_PAD + s * RPT, 8),
                                      RPT)])


def _scale_body(dpT_ref, x_ref, w_ref, y_ref, dinv_ref):
    deg = dpT_ref[:, 0:1] + dpT_ref[:, 1:2] + 1.0
    dinv = lax.rsqrt(deg)
    dinv_ref[...] = dinv
    y_ref[...] = dinv * jnp.dot(x_ref[...], w_ref[...],
                                preferred_element_type=jnp.float32)


_scale_call = pl.pallas_call(
    _scale_body,
    out_shape=[
        jax.ShapeDtypeStruct((N_PAD, D), jnp.float32),
        jax.ShapeDtypeStruct((N_PAD, 1), jnp.float32),
    ],
)


@functools.partial(
    pl.kernel,
    mesh=_mesh,
    out_type=jax.ShapeDtypeStruct((NC * N_PAD, D), jnp.float32),
    scratch_types=[
        pltpu.VMEM((CHUNK,), jnp.int32),
        pltpu.VMEM((CHUNK,), jnp.int32),
        pltpu.VMEM((CHUNK, D), jnp.float32),
        pltpu.SemaphoreType.DMA,
        pltpu.VMEM_SHARED((N_PAD, D), jnp.float32),
    ],
)
def _agg_kernel(src_hbm, dst_hbm, y_hbm, p_hbm,
                sidx_v, didx_v, rows_v, sem, agg_sh):
    c = lax.axis_index("c")
    s = lax.axis_index("s")
    base = (c * NS + s) * EPW
    row = pl.multiple_of(s * RPT, 8)
    for i in range(CHUNK):
        for j in range(D // 16):
            rows_v[i, pl.ds(j * 16, 16)] = jnp.zeros((16,), jnp.float32)
    for j in range(RPT // CHUNK):
        pltpu.sync_copy(rows_v, agg_sh.at[pl.ds(row + j * CHUNK, CHUNK)])
    plsc.subcore_barrier()

    def body(k, _):
        off = pl.multiple_of(base + k * CHUNK, 8)
        pltpu.sync_copy(src_hbm.at[pl.ds(off, CHUNK)], sidx_v)
        pltpu.sync_copy(dst_hbm.at[pl.ds(off, CHUNK)], didx_v)
        pltpu.async_copy(y_hbm.at[sidx_v], rows_v, sem).wait()
        pltpu.sync_copy(rows_v, agg_sh.at[didx_v], add=True)
        return _

    lax.fori_loop(0, NCHUNK, body, None)
    plsc.subcore_barrier()
    gbase = pl.multiple_of(c * N_PAD + s * RPT, 8)
    for j in range(RPT // CHUNK):
        pltpu.sync_copy(agg_sh.at[pl.ds(row + j * CHUNK, CHUNK)], rows_v)
        pltpu.sync_copy(rows_v, p_hbm.at[pl.ds(gbase + j * CHUNK, CHUNK)])


def _combine_body(p_ref, y_ref, dinv_ref, b_ref, o_ref):
    o_ref[...] = dinv_ref[...] * (p_ref[:N_PAD] + p_ref[N_PAD:] + y_ref[...]) \
        + b_ref[...]


_combine_call = pl.pallas_call(
    _combine_body,
    out_shape=jax.ShapeDtypeStruct((N_PAD, D), jnp.float32),
)


def kernel(x, edge_index, W, b):
    ei = edge_index.astype(jnp.int32)
    src, dst = ei[0], ei[1]
    xp = jnp.pad(x, ((0, N_PAD - N_NODES), (0, 0)))

    degp = _deg_kernel(dst)
    y, dinv = _scale_call(degp.reshape(NC, N_PAD).T, xp, W)
    p = _agg_kernel(src, dst, y)
    out = _combine_call(p, y, dinv, b.reshape(1, D))
    return out[:N_NODES]


# speedup vs baseline: 33.5003x; 1.2496x over previous
import functools

import jax
import jax.numpy as jnp
from jax import lax
from jax.experimental import pallas as pl
from jax.experimental.pallas import tpu as pltpu
from jax.experimental.pallas import tpu_sc as plsc

N_NODES = 10000
N_PAD = 10240
N_EDGES = 320000
D = 128

NC, NS = 2, 16
NW = NC * NS
CHUNK = 128
NCH = 78
NPAIR = NCH // 2
EPW = NCH * CHUNK
TBASE = NW * EPW
TAIL = 16
RPT = N_PAD // NS

_mesh = plsc.VectorSubcoreMesh(core_axis_name="c", subcore_axis_name="s")


@functools.partial(
    pl.kernel,
    mesh=_mesh,
    out_type=jax.ShapeDtypeStruct((NC * N_PAD,), jnp.float32),
    scratch_types=[
        pltpu.VMEM((CHUNK,), jnp.int32),
        pltpu.VMEM((CHUNK,), jnp.int32),
        pltpu.VMEM((TAIL,), jnp.int32),
        pltpu.VMEM((CHUNK,), jnp.float32),
        pltpu.VMEM((RPT,), jnp.float32),
        pltpu.SemaphoreType.DMA,
        pltpu.SemaphoreType.DMA,
        pltpu.VMEM_SHARED((N_PAD,), jnp.float32),
    ],
)
def _deg_kernel(dst_hbm, degp_hbm, didx0, didx1, tidx, ones_v, zb_v,
                sem0, sem1, deg_sh):
    c = lax.axis_index("c")
    s = lax.axis_index("s")
    w = c * NS + s
    base = w * EPW
    row = pl.multiple_of(s * RPT, 8)
    for i in range(CHUNK // 16):
        ones_v[pl.ds(i * 16, 16)] = jnp.ones((16,), jnp.float32)
    for i in range(RPT // 16):
        zb_v[pl.ds(i * 16, 16)] = jnp.zeros((16,), jnp.float32)
    pltpu.sync_copy(zb_v, deg_sh.at[pl.ds(row, RPT)])
    plsc.subcore_barrier()

    def load(k, buf, sem):
        off = pl.multiple_of(base + k * CHUNK, 8)
        return pltpu.async_copy(dst_hbm.at[pl.ds(off, CHUNK)], buf, sem)

    def drain(k, buf, sem):
        off = pl.multiple_of(base + k * CHUNK, 8)
        pltpu.make_async_copy(dst_hbm.at[pl.ds(off, CHUNK)], buf, sem).wait()

    load(0, didx0, sem0)

    def body(j, _):
        k0 = 2 * j
        load(k0 + 1, didx1, sem1)
        drain(k0, didx0, sem0)
        pltpu.sync_copy(ones_v, deg_sh.at[didx0], add=True)

        @pl.when(j < NPAIR - 1)
        def _issue():
            load(k0 + 2, didx0, sem0)

        drain(k0 + 1, didx1, sem1)
        pltpu.sync_copy(ones_v, deg_sh.at[didx1], add=True)
        return _

    lax.fori_loop(0, NPAIR, body, None)
    toff = pl.multiple_of(TBASE + w * TAIL, 8)
    pltpu.sync_copy(dst_hbm.at[pl.ds(toff, TAIL)], tidx)
    pltpu.sync_copy(ones_v.at[pl.ds(0, TAIL)], deg_sh.at[tidx], add=True)
    plsc.subcore_barrier()
    pltpu.sync_copy(deg_sh.at[pl.ds(row, RPT)], zb_v)
    pltpu.sync_copy(zb_v,
                    degp_hbm.at[pl.ds(pl.multiple_of(c * N_PAD + s * RPT, 8),
                                      RPT)])


def _scale_body(dpT_ref, x_ref, w_ref, y_ref, dinv_ref):
    deg = dpT_ref[:, 0:1] + dpT_ref[:, 1:2] + 1.0
    dinv = lax.rsqrt(deg)
    dinv_ref[...] = dinv
    y_ref[...] = dinv * jnp.dot(x_ref[...], w_ref[...],
                                preferred_element_type=jnp.float32)


_scale_call = pl.pallas_call(
    _scale_body,
    out_shape=[
        jax.ShapeDtypeStruct((N_PAD, D), jnp.float32),
        jax.ShapeDtypeStruct((N_PAD, 1), jnp.float32),
    ],
)


@functools.partial(
    pl.kernel,
    mesh=_mesh,
    out_type=jax.ShapeDtypeStruct((NC * N_PAD, D), jnp.float32),
    scratch_types=[
        pltpu.VMEM((CHUNK,), jnp.int32),
        pltpu.VMEM((CHUNK,), jnp.int32),
        pltpu.VMEM((CHUNK,), jnp.int32),
        pltpu.VMEM((CHUNK,), jnp.int32),
        pltpu.VMEM((TAIL,), jnp.int32),
        pltpu.VMEM((TAIL,), jnp.int32),
        pltpu.VMEM((CHUNK, D), jnp.float32),
        pltpu.VMEM((CHUNK, D), jnp.float32),
        pltpu.VMEM((TAIL, D), jnp.float32),
        pltpu.SemaphoreType.DMA,
        pltpu.SemaphoreType.DMA,
        pltpu.VMEM_SHARED((N_PAD, D), jnp.float32),
    ],
)
def _agg_kernel(src_hbm, dst_hbm, y_hbm, p_hbm,
                sidx0, sidx1, didx0, didx1, tsidx, tdidx,
                rows0, rows1, trows, sem0, sem1, agg_sh):
    c = lax.axis_index("c")
    s = lax.axis_index("s")
    w = c * NS + s
    base = w * EPW
    row = pl.multiple_of(s * RPT, 8)
    for i in range(CHUNK):
        for j in range(D // 16):
            rows0[i, pl.ds(j * 16, 16)] = jnp.zeros((16,), jnp.float32)
    for j in range(RPT // CHUNK):
        pltpu.sync_copy(rows0, agg_sh.at[pl.ds(row + j * CHUNK, CHUNK)])
    plsc.subcore_barrier()

    def issue(k, si, di, rv, sem):
        off = pl.multiple_of(base + k * CHUNK, 8)
        pltpu.sync_copy(src_hbm.at[pl.ds(off, CHUNK)], si)
        pltpu.sync_copy(dst_hbm.at[pl.ds(off, CHUNK)], di)
        pltpu.async_copy(y_hbm.at[si], rv, sem)

    def drain_scatter(si, di, rv, sem):
        pltpu.make_async_copy(y_hbm.at[si], rv, sem).wait()
        pltpu.sync_copy(rv, agg_sh.at[di], add=True)

    issue(0, sidx0, didx0, rows0, sem0)

    def body(j, _):
        k0 = 2 * j
        issue(k0 + 1, sidx1, didx1, rows1, sem1)
        drain_scatter(sidx0, didx0, rows0, sem0)

        @pl.when(j < NPAIR - 1)
        def _issue_next():
            issue(k0 + 2, sidx0, didx0, rows0, sem0)

        drain_scatter(sidx1, didx1, rows1, sem1)
        return _

    lax.fori_loop(0, NPAIR, body, None)
    toff = pl.multiple_of(TBASE + w * TAIL, 8)
    pltpu.sync_copy(src_hbm.at[pl.ds(toff, TAIL)], tsidx)
    pltpu.sync_copy(dst_hbm.at[pl.ds(toff, TAIL)], tdidx)
    pltpu.async_copy(y_hbm.at[tsidx], trows, sem0).wait()
    pltpu.sync_copy(trows, agg_sh.at[tdidx], add=True)
    plsc.subcore_barrier()
    gbase = pl.multiple_of(c * N_PAD + s * RPT, 8)
    for j in range(RPT // CHUNK):
        pltpu.sync_copy(agg_sh.at[pl.ds(row + j * CHUNK, CHUNK)], rows0)
        pltpu.sync_copy(rows0, p_hbm.at[pl.ds(gbase + j * CHUNK, CHUNK)])


def _combine_body(p_ref, y_ref, dinv_ref, b_ref, o_ref):
    o_ref[...] = dinv_ref[...] * (p_ref[:N_PAD] + p_ref[N_PAD:] + y_ref[...]) \
        + b_ref[...]


_combine_call = pl.pallas_call(
    _combine_body,
    out_shape=jax.ShapeDtypeStruct((N_PAD, D), jnp.float32),
)


def kernel(x, edge_index, W, b):
    ei = edge_index.astype(jnp.int32)
    src, dst = ei[0], ei[1]
    xp = jnp.pad(x, ((0, N_PAD - N_NODES), (0, 0)))

    degp = _deg_kernel(dst)
    y, dinv = _scale_call(degp.reshape(NC, N_PAD).T, xp, W)
    p = _agg_kernel(src, dst, y)
    out = _combine_call(p, y, dinv, b.reshape(1, D))
    return out[:N_NODES]


# speedup vs baseline: 37.9108x; 1.1317x over previous
import functools

import jax
import jax.numpy as jnp
from jax import lax
from jax.experimental import pallas as pl
from jax.experimental.pallas import tpu as pltpu
from jax.experimental.pallas import tpu_sc as plsc

N_NODES = 10000
N_PAD = 10240
N_EDGES = 320000
D = 128

NC, NS = 2, 16
NW = NC * NS
RPT = N_PAD // NS

CH1 = 128
ROWS1 = N_EDGES // CH1
WCH1 = 80
LAST1 = ROWS1 - (NW - 1) * WCH1
LAST1_LOAD = 24

CH3 = 64
ROWS3 = N_EDGES // CH3
WCH3 = 160
LAST3 = ROWS3 - (NW - 1) * WCH3
WCHP = 80

_mesh = plsc.VectorSubcoreMesh(core_axis_name="c", subcore_axis_name="s")


@functools.partial(
    pl.kernel,
    mesh=_mesh,
    out_type=jax.ShapeDtypeStruct((NC * N_PAD,), jnp.float32),
    scratch_types=[
        pltpu.VMEM((WCH1, CH1), jnp.int32),
        pltpu.VMEM((CH1,), jnp.float32),
        pltpu.VMEM((RPT,), jnp.float32),
        pltpu.VMEM_SHARED((N_PAD,), jnp.float32),
    ],
)
def _deg_kernel(dst2_hbm, degp_hbm, didx, ones_v, zb_v, deg_sh):
    c = lax.axis_index("c")
    s = lax.axis_index("s")
    w = c * NS + s
    row = pl.multiple_of(s * RPT, 8)
    for i in range(CH1 // 16):
        ones_v[pl.ds(i * 16, 16)] = jnp.ones((16,), jnp.float32)
    for i in range(RPT // 16):
        zb_v[pl.ds(i * 16, 16)] = jnp.zeros((16,), jnp.float32)
    pltpu.sync_copy(zb_v, deg_sh.at[pl.ds(row, RPT)])
    rb = pl.multiple_of(w * WCH1, 8)
    nch = jnp.where(w == NW - 1, LAST1, WCH1)

    @pl.when(w < NW - 1)
    def _full():
        pltpu.sync_copy(dst2_hbm.at[pl.ds(rb, WCH1)], didx)

    @pl.when(w == NW - 1)
    def _part():
        pltpu.sync_copy(dst2_hbm.at[pl.ds(rb, LAST1_LOAD)],
                        didx.at[pl.ds(0, LAST1_LOAD)])

    plsc.subcore_barrier()

    def body(k, _):
        pltpu.sync_copy(ones_v, deg_sh.at[didx.at[k]], add=True)
        return _

    lax.fori_loop(0, nch, body, None)
    plsc.subcore_barrier()
    pltpu.sync_copy(deg_sh.at[pl.ds(row, RPT)], zb_v)
    pltpu.sync_copy(zb_v,
                    degp_hbm.at[pl.ds(pl.multiple_of(c * N_PAD + s * RPT, 8),
                                      RPT)])


def _mm_body(x_ref, w_ref, z_ref):
    z_ref[...] = jnp.dot(x_ref[...], w_ref[...],
                         preferred_element_type=jnp.float32)


_mm_call = pl.pallas_call(
    _mm_body,
    out_shape=jax.ShapeDtypeStruct((N_PAD, D), jnp.float32),
)


def _scale_body(dpT_ref, z_ref, y_ref, dinv_ref):
    deg = dpT_ref[:, 0:1] + dpT_ref[:, 1:2] + 1.0
    dinv = lax.rsqrt(deg)
    dinv_ref[...] = dinv
    y_ref[...] = dinv * z_ref[...]


_scale_call = pl.pallas_call(
    _scale_body,
    out_shape=[
        jax.ShapeDtypeStruct((N_PAD, D), jnp.float32),
        jax.ShapeDtypeStruct((N_PAD, 1), jnp.float32),
    ],
)


@functools.partial(
    pl.kernel,
    mesh=_mesh,
    out_type=jax.ShapeDtypeStruct((NC * N_PAD, D), jnp.float32),
    scratch_types=[
        pltpu.VMEM((WCHP, CH3), jnp.int32),
        pltpu.VMEM((WCHP, CH3), jnp.int32),
        pltpu.VMEM((CH3, D), jnp.float32),
        pltpu.VMEM((CH3, D), jnp.float32),
        pltpu.SemaphoreType.DMA,
        pltpu.SemaphoreType.DMA,
        pltpu.VMEM_SHARED((N_PAD, D), jnp.float32),
    ],
)
def _agg_kernel(src2_hbm, dst2_hbm, y_hbm, p_hbm,
                sidx, didx, rows0, rows1, sem0, sem1, agg_sh):
    c = lax.axis_index("c")
    s = lax.axis_index("s")
    w = c * NS + s
    row = pl.multiple_of(s * RPT, 8)
    for i in range(CH3):
        for j in range(D // 16):
            rows0[i, pl.ds(j * 16, 16)] = jnp.zeros((16,), jnp.float32)
    for j in range(RPT // CH3):
        pltpu.sync_copy(rows0, agg_sh.at[pl.ds(row + j * CH3, CH3)])
    plsc.subcore_barrier()

    for phase in range(2):
        pb = pl.multiple_of(w * WCH3 + phase * WCHP, 8)
        npair = jnp.where(w == NW - 1,
                          (LAST3 // 2) * (1 - phase),
                          WCHP // 2)

        @pl.when(w < NW - 1)
        def _full():
            pltpu.sync_copy(src2_hbm.at[pl.ds(pb, WCHP)], sidx)
            pltpu.sync_copy(dst2_hbm.at[pl.ds(pb, WCHP)], didx)

        if phase == 0:
            @pl.when(w == NW - 1)
            def _part():
                pltpu.sync_copy(src2_hbm.at[pl.ds(pb, LAST3)],
                                sidx.at[pl.ds(0, LAST3)])
                pltpu.sync_copy(dst2_hbm.at[pl.ds(pb, LAST3)],
                                didx.at[pl.ds(0, LAST3)])

        @pl.when(npair > 0)
        def _prologue():
            pltpu.async_copy(y_hbm.at[sidx.at[0]], rows0, sem0)

        def body(j, _):
            k0 = 2 * j
            pltpu.async_copy(y_hbm.at[sidx.at[k0 + 1]], rows1, sem1)
            pltpu.make_async_copy(y_hbm.at[sidx.at[k0]], rows0, sem0).wait()
            pltpu.sync_copy(rows0, agg_sh.at[didx.at[k0]], add=True)

            @pl.when(j < npair - 1)
            def _issue_next():
                pltpu.async_copy(y_hbm.at[sidx.at[k0 + 2]], rows0, sem0)

            pltpu.make_async_copy(y_hbm.at[sidx.at[k0 + 1]], rows1, sem1).wait()
            pltpu.sync_copy(rows1, agg_sh.at[didx.at[k0 + 1]], add=True)
            return _

        lax.fori_loop(0, npair, body, None)

    plsc.subcore_barrier()
    gbase = pl.multiple_of(c * N_PAD + s * RPT, 8)
    for j in range(RPT // CH3):
        pltpu.sync_copy(agg_sh.at[pl.ds(row + j * CH3, CH3)], rows0)
        pltpu.sync_copy(rows0, p_hbm.at[pl.ds(gbase + j * CH3, CH3)])


def _combine_body(p_ref, y_ref, dinv_ref, b_ref, o_ref):
    o_ref[...] = dinv_ref[...] * (p_ref[:N_PAD] + p_ref[N_PAD:] + y_ref[...]) \
        + b_ref[...]


_combine_call = pl.pallas_call(
    _combine_body,
    out_shape=jax.ShapeDtypeStruct((N_PAD, D), jnp.float32),
)


def kernel(x, edge_index, W, b):
    ei = edge_index.astype(jnp.int32)
    src, dst = ei[0], ei[1]
    src2 = src.reshape(ROWS3, CH3)
    dst2 = dst.reshape(ROWS3, CH3)
    dst2w = jnp.pad(dst.reshape(ROWS1, CH1), ((0, LAST1_LOAD - LAST1), (0, 0)))
    xp = jnp.pad(x, ((0, N_PAD - N_NODES), (0, 0)))

    z = _mm_call(xp, W)
    degp = _deg_kernel(dst2w)
    y, dinv = _scale_call(degp.reshape(NC, N_PAD).T, z)
    p = _agg_kernel(src2, dst2, y)
    out = _combine_call(p, y, dinv, b.reshape(1, D))
    return out[:N_NODES]


# speedup vs baseline: 43.1707x; 1.1387x over previous
import functools

import jax
import jax.numpy as jnp
from jax import lax
from jax.experimental import pallas as pl
from jax.experimental.pallas import tpu as pltpu
from jax.experimental.pallas import tpu_sc as plsc

N_NODES = 10000
N_PAD = 10240
N_EDGES = 320000
D = 128

NC, NS = 2, 16
NW = NC * NS
RPT = N_PAD // NS

CH1 = 128
ROWS1 = N_EDGES // CH1
WCH1 = 80
LAST1 = ROWS1 - (NW - 1) * WCH1
LAST1_LOAD = 24

CH3 = 128
WCH3 = 80
LAST3 = 20
LAST3_LOAD = 24
WCHP = 40

_mesh = plsc.VectorSubcoreMesh(core_axis_name="c", subcore_axis_name="s")


@functools.partial(
    pl.kernel,
    mesh=_mesh,
    out_type=jax.ShapeDtypeStruct((NC * N_PAD,), jnp.float32),
    scratch_types=[
        pltpu.VMEM((WCH1, CH1), jnp.int32),
        pltpu.VMEM((CH1,), jnp.float32),
        pltpu.VMEM((RPT,), jnp.float32),
        pltpu.VMEM_SHARED((N_PAD,), jnp.float32),
    ],
)
def _deg_kernel(dst2_hbm, degp_hbm, didx, ones_v, zb_v, deg_sh):
    c = lax.axis_index("c")
    s = lax.axis_index("s")
    w = c * NS + s
    row = pl.multiple_of(s * RPT, 8)
    for i in range(CH1 // 16):
        ones_v[pl.ds(i * 16, 16)] = jnp.ones((16,), jnp.float32)
    for i in range(RPT // 16):
        zb_v[pl.ds(i * 16, 16)] = jnp.zeros((16,), jnp.float32)
    pltpu.sync_copy(zb_v, deg_sh.at[pl.ds(row, RPT)])
    rb = pl.multiple_of(w * WCH1, 8)
    nch = jnp.where(w == NW - 1, LAST1, WCH1)

    @pl.when(w < NW - 1)
    def _full():
        pltpu.sync_copy(dst2_hbm.at[pl.ds(rb, WCH1)], didx)

    @pl.when(w == NW - 1)
    def _part():
        pltpu.sync_copy(dst2_hbm.at[pl.ds(rb, LAST1_LOAD)],
                        didx.at[pl.ds(0, LAST1_LOAD)])

    plsc.subcore_barrier()

    def body(k, _):
        pltpu.sync_copy(ones_v, deg_sh.at[didx.at[k]], add=True)
        return _

    lax.fori_loop(0, nch, body, None)
    plsc.subcore_barrier()
    pltpu.sync_copy(deg_sh.at[pl.ds(row, RPT)], zb_v)
    pltpu.sync_copy(zb_v,
                    degp_hbm.at[pl.ds(pl.multiple_of(c * N_PAD + s * RPT, 8),
                                      RPT)])


def _mm_body(x_ref, w_ref, z_ref):
    z_ref[...] = jnp.dot(x_ref[...], w_ref[...],
                         preferred_element_type=jnp.float32)


_mm_call = pl.pallas_call(
    _mm_body,
    out_shape=jax.ShapeDtypeStruct((N_PAD, D), jnp.float32),
)


def _scale_body(dpT_ref, z_ref, y_ref, dinv_ref):
    deg = dpT_ref[:, 0:1] + dpT_ref[:, 1:2] + 1.0
    dinv = lax.rsqrt(deg)
    dinv_ref[...] = dinv
    y_ref[...] = dinv * z_ref[...]


_scale_call = pl.pallas_call(
    _scale_body,
    out_shape=[
        jax.ShapeDtypeStruct((N_PAD, D), jnp.float32),
        jax.ShapeDtypeStruct((N_PAD, 1), jnp.float32),
    ],
)


@functools.partial(
    pl.kernel,
    mesh=_mesh,
    out_type=jax.ShapeDtypeStruct((NC * N_PAD, D), jnp.float32),
    scratch_types=[
        pltpu.VMEM((WCHP, CH3), jnp.int32),
        pltpu.VMEM((WCHP, CH3), jnp.int32),
        pltpu.VMEM((CH3, D), jnp.float32),
        pltpu.VMEM((CH3, D), jnp.float32),
        pltpu.SemaphoreType.DMA,
        pltpu.SemaphoreType.DMA,
        pltpu.VMEM_SHARED((N_PAD, D), jnp.float32),
    ],
)
def _agg_kernel(src2_hbm, dst2_hbm, y_hbm, p_hbm,
                sidx, didx, rows0, rows1, sem0, sem1, agg_sh):
    c = lax.axis_index("c")
    s = lax.axis_index("s")
    w = c * NS + s
    row = pl.multiple_of(s * RPT, 8)
    for i in range(CH3):
        for j in range(D // 16):
            rows0[i, pl.ds(j * 16, 16)] = jnp.zeros((16,), jnp.float32)
    for j in range(RPT // CH3):
        pltpu.sync_copy(rows0, agg_sh.at[pl.ds(row + j * CH3, CH3)])
    plsc.subcore_barrier()

    for phase in range(2):
        pb = pl.multiple_of(w * WCH3 + phase * WCHP, 8)
        npair = jnp.where(w == NW - 1,
                          (LAST3 // 2) * (1 - phase),
                          WCHP // 2)

        @pl.when(w < NW - 1)
        def _full():
            pltpu.sync_copy(src2_hbm.at[pl.ds(pb, WCHP)], sidx)
            pltpu.sync_copy(dst2_hbm.at[pl.ds(pb, WCHP)], didx)

        if phase == 0:
            @pl.when(w == NW - 1)
            def _part():
                pltpu.sync_copy(src2_hbm.at[pl.ds(pb, LAST3_LOAD)],
                                sidx.at[pl.ds(0, LAST3_LOAD)])
                pltpu.sync_copy(dst2_hbm.at[pl.ds(pb, LAST3_LOAD)],
                                didx.at[pl.ds(0, LAST3_LOAD)])

        @pl.when(npair > 0)
        def _prologue():
            pltpu.async_copy(y_hbm.at[sidx.at[0]], rows0, sem0)

        def body(j, _):
            k0 = 2 * j
            pltpu.async_copy(y_hbm.at[sidx.at[k0 + 1]], rows1, sem1)
            pltpu.make_async_copy(y_hbm.at[sidx.at[k0]], rows0, sem0).wait()
            pltpu.sync_copy(rows0, agg_sh.at[didx.at[k0]], add=True)

            @pl.when(j < npair - 1)
            def _issue_next():
                pltpu.async_copy(y_hbm.at[sidx.at[k0 + 2]], rows0, sem0)

            pltpu.make_async_copy(y_hbm.at[sidx.at[k0 + 1]], rows1, sem1).wait()
            pltpu.sync_copy(rows1, agg_sh.at[didx.at[k0 + 1]], add=True)
            return _

        lax.fori_loop(0, npair, body, None)

    plsc.subcore_barrier()
    gbase = pl.multiple_of(c * N_PAD + s * RPT, 8)
    for j in range(RPT // CH3):
        pltpu.sync_copy(agg_sh.at[pl.ds(row + j * CH3, CH3)], rows0)
        pltpu.sync_copy(rows0, p_hbm.at[pl.ds(gbase + j * CH3, CH3)])


def _combine_body(p_ref, y_ref, dinv_ref, b_ref, o_ref):
    o_ref[...] = dinv_ref[...] * (p_ref[:N_PAD] + p_ref[N_PAD:] + y_ref[...]) \
        + b_ref[...]


_combine_call = pl.pallas_call(
    _combine_body,
    out_shape=jax.ShapeDtypeStruct((N_PAD, D), jnp.float32),
)


def kernel(x, edge_index, W, b):
    ei = edge_index.astype(jnp.int32)
    src, dst = ei[0], ei[1]
    src2 = jnp.pad(src.reshape(ROWS1, CH1), ((0, LAST1_LOAD - LAST1), (0, 0)))
    dst2w = jnp.pad(dst.reshape(ROWS1, CH1), ((0, LAST1_LOAD - LAST1), (0, 0)))
    xp = jnp.pad(x, ((0, N_PAD - N_NODES), (0, 0)))

    z = _mm_call(xp, W)
    degp = _deg_kernel(dst2w)
    y, dinv = _scale_call(degp.reshape(NC, N_PAD).T, z)
    p = _agg_kernel(src2, dst2w, y)
    out = _combine_call(p, y, dinv, b.reshape(1, D))
    return out[:N_NODES]


# speedup vs baseline: 44.1969x; 1.0238x over previous
import functools

import jax
import jax.numpy as jnp
from jax import lax
from jax.experimental import pallas as pl
from jax.experimental.pallas import tpu as pltpu
from jax.experimental.pallas import tpu_sc as plsc

N_NODES = 10000
N_PAD = 10240
N_EDGES = 320000
D = 128

NC, NS = 2, 16
NW = NC * NS
RPT = N_PAD // NS

CH1 = 128
ROWS1 = N_EDGES // CH1
WCH1 = 80
LAST1 = ROWS1 - (NW - 1) * WCH1
LAST1_LOAD = 24

CH3 = 128
WCH3 = 80
LAST3 = 20
LAST3_LOAD = 24
WCHP = 40

_mesh = plsc.VectorSubcoreMesh(core_axis_name="c", subcore_axis_name="s")


@functools.partial(
    pl.kernel,
    mesh=_mesh,
    out_type=jax.ShapeDtypeStruct((NC * N_PAD,), jnp.float32),
    scratch_types=[
        pltpu.VMEM((WCH1, CH1), jnp.int32),
        pltpu.VMEM((CH1,), jnp.float32),
        pltpu.VMEM((RPT,), jnp.float32),
        pltpu.VMEM_SHARED((N_PAD,), jnp.float32),
    ],
)
def _deg_kernel(dst2_hbm, degp_hbm, didx, ones_v, zb_v, deg_sh):
    c = lax.axis_index("c")
    s = lax.axis_index("s")
    w = c * NS + s
    row = pl.multiple_of(s * RPT, 8)
    for i in range(CH1 // 16):
        ones_v[pl.ds(i * 16, 16)] = jnp.ones((16,), jnp.float32)
    for i in range(RPT // 16):
        zb_v[pl.ds(i * 16, 16)] = jnp.zeros((16,), jnp.float32)
    pltpu.sync_copy(zb_v, deg_sh.at[pl.ds(row, RPT)])
    rb = pl.multiple_of(w * WCH1, 8)
    nch = jnp.where(w == NW - 1, LAST1, WCH1)

    @pl.when(w < NW - 1)
    def _full():
        pltpu.sync_copy(dst2_hbm.at[pl.ds(rb, WCH1)], didx)

    @pl.when(w == NW - 1)
    def _part():
        pltpu.sync_copy(dst2_hbm.at[pl.ds(rb, LAST1_LOAD)],
                        didx.at[pl.ds(0, LAST1_LOAD)])

    plsc.subcore_barrier()

    def body(k, _):
        pltpu.sync_copy(ones_v, deg_sh.at[didx.at[k]], add=True)
        return _

    lax.fori_loop(0, nch, body, None)
    plsc.subcore_barrier()
    pltpu.sync_copy(deg_sh.at[pl.ds(row, RPT)], zb_v)
    pltpu.sync_copy(zb_v,
                    degp_hbm.at[pl.ds(pl.multiple_of(c * N_PAD + s * RPT, 8),
                                      RPT)])


def _mm_body(x_ref, w_ref, z_ref):
    z_ref[:N_NODES] = jnp.dot(x_ref[...], w_ref[...],
                              preferred_element_type=jnp.float32)


_mm_call = pl.pallas_call(
    _mm_body,
    out_shape=jax.ShapeDtypeStruct((N_PAD, D), jnp.float32),
)


def _scale_body(dpT_ref, z_ref, y_ref, dinv_ref):
    deg = dpT_ref[:, 0:1] + dpT_ref[:, 1:2] + 1.0
    dinv = lax.rsqrt(deg)
    dinv_ref[...] = dinv
    y_ref[...] = dinv * z_ref[...]


_scale_call = pl.pallas_call(
    _scale_body,
    out_shape=[
        jax.ShapeDtypeStruct((N_PAD, D), jnp.float32),
        jax.ShapeDtypeStruct((N_PAD, 1), jnp.float32),
    ],
)


@functools.partial(
    pl.kernel,
    mesh=_mesh,
    out_type=jax.ShapeDtypeStruct((NC * N_PAD, D), jnp.float32),
    scratch_types=[
        pltpu.VMEM((WCHP, CH3), jnp.int32),
        pltpu.VMEM((WCHP, CH3), jnp.int32),
        pltpu.VMEM((CH3, D), jnp.float32),
        pltpu.VMEM((CH3, D), jnp.float32),
        pltpu.SemaphoreType.DMA,
        pltpu.SemaphoreType.DMA,
        pltpu.VMEM_SHARED((N_PAD, D), jnp.float32),
    ],
)
def _agg_kernel(src2_hbm, dst2_hbm, y_hbm, p_hbm,
                sidx, didx, rows0, rows1, sem0, sem1, agg_sh):
    c = lax.axis_index("c")
    s = lax.axis_index("s")
    w = c * NS + s
    row = pl.multiple_of(s * RPT, 8)
    for i in range(CH3):
        for j in range(D // 16):
            rows0[i, pl.ds(j * 16, 16)] = jnp.zeros((16,), jnp.float32)
    for j in range(RPT // CH3):
        pltpu.sync_copy(rows0, agg_sh.at[pl.ds(row + j * CH3, CH3)])
    plsc.subcore_barrier()

    for phase in range(2):
        pb = pl.multiple_of(w * WCH3 + phase * WCHP, 8)
        npair = jnp.where(w == NW - 1,
                          (LAST3 // 2) * (1 - phase),
                          WCHP // 2)

        @pl.when(w < NW - 1)
        def _full():
            pltpu.sync_copy(src2_hbm.at[pl.ds(pb, WCHP)], sidx)
            pltpu.sync_copy(dst2_hbm.at[pl.ds(pb, WCHP)], didx)

        if phase == 0:
            @pl.when(w == NW - 1)
            def _part():
                pltpu.sync_copy(src2_hbm.at[pl.ds(pb, LAST3_LOAD)],
                                sidx.at[pl.ds(0, LAST3_LOAD)])
                pltpu.sync_copy(dst2_hbm.at[pl.ds(pb, LAST3_LOAD)],
                                didx.at[pl.ds(0, LAST3_LOAD)])

        @pl.when(npair > 0)
        def _prologue():
            pltpu.async_copy(y_hbm.at[sidx.at[0]], rows0, sem0)

        def body(j, _):
            k0 = 2 * j
            pltpu.async_copy(y_hbm.at[sidx.at[k0 + 1]], rows1, sem1)
            pltpu.make_async_copy(y_hbm.at[sidx.at[k0]], rows0, sem0).wait()
            pltpu.sync_copy(rows0, agg_sh.at[didx.at[k0]], add=True)

            @pl.when(j < npair - 1)
            def _issue_next():
                pltpu.async_copy(y_hbm.at[sidx.at[k0 + 2]], rows0, sem0)

            pltpu.make_async_copy(y_hbm.at[sidx.at[k0 + 1]], rows1, sem1).wait()
            pltpu.sync_copy(rows1, agg_sh.at[didx.at[k0 + 1]], add=True)
            return _

        lax.fori_loop(0, npair, body, None)

    plsc.subcore_barrier()
    gbase = pl.multiple_of(c * N_PAD + s * RPT, 8)
    for j in range(RPT // CH3):
        pltpu.sync_copy(agg_sh.at[pl.ds(row + j * CH3, CH3)], rows0)
        pltpu.sync_copy(rows0, p_hbm.at[pl.ds(gbase + j * CH3, CH3)])


def _combine_body(p_ref, y_ref, dinv_ref, b_ref, o_ref):
    o_ref[...] = dinv_ref[:N_NODES] * (
        p_ref[:N_NODES] + p_ref[N_PAD:N_PAD + N_NODES] + y_ref[:N_NODES]
    ) + b_ref[...]


_combine_call = pl.pallas_call(
    _combine_body,
    out_shape=jax.ShapeDtypeStruct((N_NODES, D), jnp.float32),
)


def kernel(x, edge_index, W, b):
    ei = edge_index.astype(jnp.int32)
    src, dst = ei[0], ei[1]
    src2 = jnp.pad(src.reshape(ROWS1, CH1), ((0, LAST1_LOAD - LAST1), (0, 0)))
    dst2w = jnp.pad(dst.reshape(ROWS1, CH1), ((0, LAST1_LOAD - LAST1), (0, 0)))
    z = _mm_call(x, W)
    degp = _deg_kernel(dst2w)
    y, dinv = _scale_call(degp.reshape(NC, N_PAD).T, z)
    p = _agg_kernel(src2, dst2w, y)
    return _combine_call(p, y, dinv, b.reshape(1, D))
